# baseline (speedup 1.0000x reference)
import jax
import jax.numpy as jnp
from jax import lax
from jax.experimental import pallas as pl
from jax.experimental.pallas import tpu as pltpu
from jax.experimental.pallas import tpu_sc as plsc

BATCH = 16384
EMBED_DIM = 32
LANES = 16
NUM_CORES = 2
NUM_SUBCORES = 16
NUM_WORKERS = NUM_CORES * NUM_SUBCORES
B_PER_W = BATCH // NUM_WORKERS
CHUNK = 128
NCHUNKS = B_PER_W // CHUNK
GROUPS = B_PER_W // LANES


def _gmf_body(users_hbm, items_hbm, utabT_hbm, itabT_hbm, params_hbm,
              out_hbm, uidx_v, iidx_v, ucols_v, icols_v, wv, outv,
              sem_u, sem_i):
    wid = lax.axis_index("s") * NUM_CORES + lax.axis_index("c")
    base = wid * B_PER_W

    for j in range(NCHUNKS):
        pltpu.sync_copy(users_hbm.at[pl.ds(base + j * CHUNK, CHUNK)],
                        uidx_v.at[j])
        pltpu.sync_copy(items_hbm.at[pl.ds(base + j * CHUNK, CHUNK)],
                        iidx_v.at[j])
    pltpu.sync_copy(params_hbm, wv)

    copies = []
    for d in range(EMBED_DIM):
        for j in range(NCHUNKS):
            copies.append(pltpu.async_copy(
                utabT_hbm.at[d].at[uidx_v.at[j]],
                ucols_v.at[d, pl.ds(j * CHUNK, CHUNK)], sem_u))
            copies.append(pltpu.async_copy(
                itabT_hbm.at[d].at[iidx_v.at[j]],
                icols_v.at[d, pl.ds(j * CHUNK, CHUNK)], sem_i))
    for c in copies:
        c.wait()

    w_lo = wv[pl.ds(0, LANES)]
    w_hi = wv[pl.ds(LANES, LANES)]
    bias = wv[pl.ds(2 * LANES, LANES)][0]

    def group(g, carry):
        acc = jnp.full((LANES,), bias, dtype=jnp.float32)
        for d in range(EMBED_DIM):
            uc = ucols_v[d, pl.ds(g * LANES, LANES)]
            ic = icols_v[d, pl.ds(g * LANES, LANES)]
            wd = w_lo[d] if d < LANES else w_hi[d - LANES]
            acc = acc + uc * ic * wd
        outv[pl.ds(g * LANES, LANES)] = acc
        return carry

    lax.fori_loop(0, GROUPS, group, 0)

    pltpu.sync_copy(outv, out_hbm.at[pl.ds(base, B_PER_W)])


@jax.jit
def _gmf(users, items, utabT, itabT, params):
    mesh = plsc.VectorSubcoreMesh(core_axis_name="c", subcore_axis_name="s")
    return pl.kernel(
        _gmf_body,
        out_type=jax.ShapeDtypeStruct((BATCH,), jnp.float32),
        mesh=mesh,
        compiler_params=pltpu.CompilerParams(
            needs_layout_passes=False, use_tc_tiling_on_sc=False),
        scratch_types=[
            pltpu.VMEM((NCHUNKS, CHUNK), jnp.int32),
            pltpu.VMEM((NCHUNKS, CHUNK), jnp.int32),
            pltpu.VMEM((EMBED_DIM, B_PER_W), jnp.float32),
            pltpu.VMEM((EMBED_DIM, B_PER_W), jnp.float32),
            pltpu.VMEM((48,), jnp.float32),
            pltpu.VMEM((B_PER_W,), jnp.float32),
            pltpu.SemaphoreType.DMA,
            pltpu.SemaphoreType.DMA,
        ],
    )(users, items, utabT, itabT, params)


def kernel(users, items, user_table, item_table, W, b):
    params = jnp.zeros((48,), jnp.float32)
    params = params.at[:EMBED_DIM].set(W.reshape(-1))
    params = params.at[EMBED_DIM:EMBED_DIM + 1].set(b)
    return _gmf(users, items, user_table.T, item_table.T, params)


# speedup vs baseline: 5.5724x; 5.5724x over previous
import jax
import jax.numpy as jnp
from jax import lax
from jax.experimental import pallas as pl
from jax.experimental.pallas import tpu as pltpu
from jax.experimental.pallas import tpu_sc as plsc

BATCH = 16384
EMBED_DIM = 32
LANES = 16
NUM_CORES = 2
NUM_SUBCORES = 16
NUM_WORKERS = NUM_CORES * NUM_SUBCORES
B_PER_W = BATCH // NUM_WORKERS
CHUNK = 128
NCHUNKS = B_PER_W // CHUNK
GPC = CHUNK // LANES
PACK = 128 // EMBED_DIM


def _gmf_body(users_hbm, items_hbm, utab_hbm, itab_hbm, params_hbm,
              out_hbm, uidx_v, iidx_v, uq_v, iq_v, urows_v, irows_v, wv,
              outv, sem_u, sem_i):
    wid = lax.axis_index("s") * NUM_CORES + lax.axis_index("c")
    base = wid * B_PER_W

    for j in range(NCHUNKS):
        pltpu.sync_copy(users_hbm.at[pl.ds(base + j * CHUNK, CHUNK)],
                        uidx_v.at[j])
        pltpu.sync_copy(items_hbm.at[pl.ds(base + j * CHUNK, CHUNK)],
                        iidx_v.at[j])
    pltpu.sync_copy(params_hbm, wv)

    for j in range(NCHUNKS):
        def lineids(k, carry, j=j):
            sl = pl.ds(k * LANES, LANES)
            for idx_ref, q_ref in ((uidx_v, uq_v), (iidx_v, iq_v)):
                q_ref.at[j][sl] = lax.shift_right_logical(
                    idx_ref.at[j][sl], 2)
            return carry

        lax.fori_loop(0, CHUNK // LANES, lineids, 0)

    w_lo = wv[pl.ds(0, LANES)]
    w_hi = wv[pl.ds(LANES, LANES)]
    bias = wv[pl.ds(2 * LANES, LANES)][0]
    lane = lax.iota(jnp.int32, LANES)

    for j in range(NCHUNKS):
        cu = pltpu.async_copy(utab_hbm.at[uq_v.at[j]], urows_v, sem_u)
        ci = pltpu.async_copy(itab_hbm.at[iq_v.at[j]], irows_v, sem_i)
        cu.wait()
        ci.wait()

        def group(g, carry, j=j):
            rowids = g * LANES + lane
            sl = pl.ds(g * LANES, LANES)
            uo = (uidx_v.at[j][sl] & 3) * EMBED_DIM
            io = (iidx_v.at[j][sl] & 3) * EMBED_DIM
            acc = jnp.full((LANES,), bias, dtype=jnp.float32)
            for d in range(EMBED_DIM):
                uc = plsc.load_gather(urows_v, [rowids, uo + d])
                ic = plsc.load_gather(irows_v, [rowids, io + d])
                wd = w_lo[d] if d < LANES else w_hi[d - LANES]
                acc = acc + uc * ic * wd
            outv[pl.ds(j * CHUNK + g * LANES, LANES)] = acc
            return carry

        lax.fori_loop(0, GPC, group, 0)

    pltpu.sync_copy(outv, out_hbm.at[pl.ds(base, B_PER_W)])


@jax.jit
def _gmf(users, items, utab, itab, params):
    mesh = plsc.VectorSubcoreMesh(core_axis_name="c", subcore_axis_name="s")
    return pl.kernel(
        _gmf_body,
        out_type=jax.ShapeDtypeStruct((BATCH,), jnp.float32),
        mesh=mesh,
        compiler_params=pltpu.CompilerParams(needs_layout_passes=False),
        scratch_types=[
            pltpu.VMEM((NCHUNKS, CHUNK), jnp.int32),
            pltpu.VMEM((NCHUNKS, CHUNK), jnp.int32),
            pltpu.VMEM((NCHUNKS, CHUNK), jnp.int32),
            pltpu.VMEM((NCHUNKS, CHUNK), jnp.int32),
            pltpu.VMEM((CHUNK, 128), jnp.float32),
            pltpu.VMEM((CHUNK, 128), jnp.float32),
            pltpu.VMEM((48,), jnp.float32),
            pltpu.VMEM((B_PER_W,), jnp.float32),
            pltpu.SemaphoreType.DMA,
            pltpu.SemaphoreType.DMA,
        ],
    )(users, items, utab, itab, params)


def kernel(users, items, user_table, item_table, W, b):
    params = jnp.zeros((48,), jnp.float32)
    params = params.at[:EMBED_DIM].set(W.reshape(-1))
    params = params.at[EMBED_DIM:EMBED_DIM + 1].set(b)
    utab = user_table.reshape(-1, 128)
    itab = item_table.reshape(-1, 128)
    return _gmf(users, items, utab, itab, params)


# speedup vs baseline: 9.3884x; 1.6848x over previous
import jax
import jax.numpy as jnp
from jax import lax
from jax.experimental import pallas as pl
from jax.experimental.pallas import tpu as pltpu
from jax.experimental.pallas import tpu_sc as plsc

BATCH = 16384
EMBED_DIM = 32
LANES = 16
NUM_CORES = 2
NUM_SUBCORES = 16
NUM_WORKERS = NUM_CORES * NUM_SUBCORES
B_PER_W = BATCH // NUM_WORKERS
CHUNK = 128
NCHUNKS = B_PER_W // CHUNK
GPC = CHUNK // LANES

NROWS = 1000000
WBLK = 8192
PACK_GRID = (NROWS + WBLK - 1) // WBLK
NLINES = PACK_GRID * (WBLK // 4)


def _pack_body(in_ref, out_ref):
    eye = jnp.eye(EMBED_DIM, dtype=jnp.float32)
    for sb in range(WBLK // 4096):
        for q in range(4):
            tr = lax.dot_general(
                in_ref[:, sb * 4096 + q * 1024:sb * 4096 + (q + 1) * 1024],
                eye, (((0,), (0,)), ((), ())),
                preferred_element_type=jnp.float32)
            out_ref[sb * 1024:(sb + 1) * 1024,
                    q * EMBED_DIM:(q + 1) * EMBED_DIM] = tr


_pack = pl.pallas_call(
    _pack_body,
    grid=(PACK_GRID,),
    compiler_params=pltpu.CompilerParams(
        fuse_transposed_lhs_in_matmul=True),
    in_specs=[pl.BlockSpec((EMBED_DIM, WBLK), lambda i: (0, i))],
    out_specs=pl.BlockSpec((WBLK // 4, 128), lambda i: (i, 0)),
    out_shape=jax.ShapeDtypeStruct((NLINES, 128), jnp.float32),
)


def _gmf_body(users_hbm, items_hbm, utab_hbm, itab_hbm, params_hbm,
              out_hbm, uidx_v, iidx_v, uq_v, iq_v, urows_v, irows_v, wv,
              outv, sem_u, sem_i):
    wid = lax.axis_index("s") * NUM_CORES + lax.axis_index("c")
    base = wid * B_PER_W

    for j in range(NCHUNKS):
        pltpu.sync_copy(users_hbm.at[pl.ds(base + j * CHUNK, CHUNK)],
                        uidx_v.at[j])
        pltpu.sync_copy(items_hbm.at[pl.ds(base + j * CHUNK, CHUNK)],
                        iidx_v.at[j])
    pltpu.sync_copy(params_hbm, wv)

    for j in range(NCHUNKS):
        def lineids(k, carry, j=j):
            sl = pl.ds(k * LANES, LANES)
            for idx_ref, q_ref in ((uidx_v, uq_v), (iidx_v, iq_v)):
                i = idx_ref.at[j][sl]
                q_ref.at[j][sl] = (
                    lax.shift_left(lax.shift_right_logical(i, 12), 10)
                    | (i & 1023))
            return carry

        lax.fori_loop(0, CHUNK // LANES, lineids, 0)

    w_lo = wv[pl.ds(0, LANES)]
    w_hi = wv[pl.ds(LANES, LANES)]
    bias = wv[pl.ds(2 * LANES, LANES)][0]
    lane = lax.iota(jnp.int32, LANES)

    for j in range(NCHUNKS):
        cu = pltpu.async_copy(utab_hbm.at[uq_v.at[j]], urows_v, sem_u)
        ci = pltpu.async_copy(itab_hbm.at[iq_v.at[j]], irows_v, sem_i)
        cu.wait()
        ci.wait()

        def group(g, carry, j=j):
            rowids = g * LANES + lane
            sl = pl.ds(g * LANES, LANES)
            uo = (lax.shift_right_logical(uidx_v.at[j][sl], 10) & 3)
            io = (lax.shift_right_logical(iidx_v.at[j][sl], 10) & 3)
            uo = uo * EMBED_DIM
            io = io * EMBED_DIM
            acc = jnp.full((LANES,), bias, dtype=jnp.float32)
            for d in range(EMBED_DIM):
                uc = plsc.load_gather(urows_v, [rowids, uo + d])
                ic = plsc.load_gather(irows_v, [rowids, io + d])
                wd = w_lo[d] if d < LANES else w_hi[d - LANES]
                acc = acc + uc * ic * wd
            outv[pl.ds(j * CHUNK + g * LANES, LANES)] = acc
            return carry

        lax.fori_loop(0, GPC, group, 0)

    pltpu.sync_copy(outv, out_hbm.at[pl.ds(base, B_PER_W)])


@jax.jit
def _gmf(users, items, user_table, item_table, params):
    utab = _pack(user_table.T)
    itab = _pack(item_table.T)
    mesh = plsc.VectorSubcoreMesh(core_axis_name="c", subcore_axis_name="s")
    return pl.kernel(
        _gmf_body,
        out_type=jax.ShapeDtypeStruct((BATCH,), jnp.float32),
        mesh=mesh,
        compiler_params=pltpu.CompilerParams(needs_layout_passes=False),
        scratch_types=[
            pltpu.VMEM((NCHUNKS, CHUNK), jnp.int32),
            pltpu.VMEM((NCHUNKS, CHUNK), jnp.int32),
            pltpu.VMEM((NCHUNKS, CHUNK), jnp.int32),
            pltpu.VMEM((NCHUNKS, CHUNK), jnp.int32),
            pltpu.VMEM((CHUNK, 128), jnp.float32),
            pltpu.VMEM((CHUNK, 128), jnp.float32),
            pltpu.VMEM((48,), jnp.float32),
            pltpu.VMEM((B_PER_W,), jnp.float32),
            pltpu.SemaphoreType.DMA,
            pltpu.SemaphoreType.DMA,
        ],
    )(users, items, utab, itab, params)


def kernel(users, items, user_table, item_table, W, b):
    params = jnp.zeros((48,), jnp.float32)
    params = params.at[:EMBED_DIM].set(W.reshape(-1))
    params = params.at[EMBED_DIM:EMBED_DIM + 1].set(b)
    return _gmf(users, items, user_table, item_table, params)
